# baseline (speedup 1.0000x reference)
import jax
import jax.numpy as jnp
from jax import lax
from jax.experimental import pallas as pl
from jax.experimental.pallas import tpu as pltpu
from jax.experimental.pallas import tpu_sc as plsc

N = 32768
B = 16
D = 128
GC = 512
NEG = -1e30

NCORE = 2
NSUB = 16
NW = NCORE * NSUB
TPW = N // NW
CHT = 128
NCH = TPW // CHT
GRP = CHT // 16


def _sc_body(x_hbm, qt_hbm, w_hbm, out_hbm,
             w_v, xb0, xb1, qb0, qb1, lb, sx0, sx1, sq0, sq1):
    wid = lax.axis_index("s") * NCORE + lax.axis_index("c")
    tok0 = wid * TPW

    pltpu.sync_copy(w_hbm, w_v)

    xbufs = (xb0, xb1)
    qbufs = (qb0, qb1)
    sxs = (sx0, sx1)
    sqs = (sq0, sq1)

    def issue(c, b):
        pltpu.async_copy(x_hbm.at[pl.ds((tok0 + c * CHT) * D, CHT * D)], xbufs[b], sxs[b])
        pltpu.async_copy(qt_hbm.at[:, pl.ds(tok0 + c * CHT, CHT)], qbufs[b], sqs[b])

    issue(0, 0)
    issue(1, 1)

    lane = jnp.arange(16, dtype=jnp.int32)

    def outer(g, carry):
        for b in range(2):
            c = g * 2 + b
            pltpu.make_async_copy(x_hbm.at[pl.ds(0, CHT * D)], xbufs[b], sxs[b]).wait()
            pltpu.make_async_copy(qt_hbm.at[:, pl.ds(0, CHT)], qbufs[b], sqs[b]).wait()

            def grp_body(t, carry2, b=b, c=c):
                tv = (t * 16 + lane) * D
                qvs = [qbufs[b][l, pl.ds(t * 16, 16)] * D for l in range(3)]
                acc = [jnp.zeros((16,), jnp.float32) for _ in range(3)]
                for d in range(D):
                    xv = plsc.load_gather(xbufs[b], [tv + d])
                    for l in range(3):
                        wv = plsc.load_gather(w_v, [qvs[l] + d])
                        acc[l] = acc[l] + xv * wv
                for l in range(3):
                    lb[l, pl.ds(c * CHT + t * 16, 16)] = acc[l]
                return carry2

            lax.fori_loop(0, GRP, grp_body, 0)

            @pl.when(c + 2 < NCH)
            def _prefetch(b=b, c=c):
                issue(c + 2, b)
        return carry

    lax.fori_loop(0, NCH // 2, outer, 0)
    pltpu.sync_copy(lb, out_hbm.at[:, pl.ds(tok0, TPW)])


def _sc_logits(x, q_t, attn_weight):
    return pl.kernel(
        _sc_body,
        out_type=jax.ShapeDtypeStruct((3, N), jnp.float32),
        mesh=plsc.VectorSubcoreMesh(core_axis_name="c", subcore_axis_name="s"),
        compiler_params=pltpu.CompilerParams(needs_layout_passes=False),
        scratch_types=[
            pltpu.VMEM((GC * D,), jnp.float32),
            pltpu.VMEM((CHT * D,), jnp.float32),
            pltpu.VMEM((CHT * D,), jnp.float32),
            pltpu.VMEM((3, CHT), jnp.int32),
            pltpu.VMEM((3, CHT), jnp.int32),
            pltpu.VMEM((3, TPW), jnp.float32),
            pltpu.SemaphoreType.DMA,
            pltpu.SemaphoreType.DMA,
            pltpu.SemaphoreType.DMA,
            pltpu.SemaphoreType.DMA,
        ],
    )(x, q_t, attn_weight)


CH = 2048
NCHUNK = N // CH


def _tc_body(lg_ref, x_ref, cu_ref, out_ref, m_ref, d_ref, a_ref):
    i = pl.program_id(0)

    @pl.when(i == 0)
    def _init():
        m_ref[...] = jnp.full((3, B), NEG, jnp.float32)
        d_ref[...] = jnp.zeros((3, B), jnp.float32)
        a_ref[...] = jnp.zeros((3, B, D), jnp.float32)

    x_c = x_ref[...]
    tok = i * CH + lax.broadcasted_iota(jnp.int32, (1, CH), 1)
    cu = jnp.stack([cu_ref[bb] for bb in range(B + 1)])
    lo = cu[:B][:, None]
    hi = cu[1:][:, None]
    mask = (tok >= lo) & (tok < hi)
    lg = lg_ref[...]

    for l in range(3):
        logit = lg[l:l + 1, :]
        lmask = jnp.where(mask, logit, NEG)
        cmax = jnp.max(lmask, axis=1)
        m_old = m_ref[l, :]
        m_new = jnp.maximum(m_old, cmax)
        scale = jnp.exp(m_old - m_new)
        e = jnp.where(mask, jnp.exp(logit - m_new[:, None]), 0.0)
        d_new = d_ref[l, :] * scale + jnp.sum(e, axis=1)
        contrib = jnp.dot(e, x_c, preferred_element_type=jnp.float32)
        a_new = a_ref[l] * scale[:, None] + contrib
        m_ref[l, :] = m_new
        d_ref[l, :] = d_new
        a_ref[l] = a_new

        @pl.when(i == NCHUNK - 1)
        def _fin():
            denom = d_new[:, None]
            out_ref[l] = jnp.where(denom > 0.0, a_new / denom, 0.0)


def _tc_pool(logits, x, cu_seqlens):
    return pl.pallas_call(
        _tc_body,
        grid=(NCHUNK,),
        in_specs=[
            pl.BlockSpec((3, CH), lambda i: (0, i)),
            pl.BlockSpec((CH, D), lambda i: (i, 0)),
            pl.BlockSpec(memory_space=pltpu.SMEM),
        ],
        out_specs=pl.BlockSpec((3, B, D), lambda i: (0, 0, 0)),
        out_shape=jax.ShapeDtypeStruct((3, B, D), jnp.float32),
        scratch_shapes=[
            pltpu.VMEM((3, B), jnp.float32),
            pltpu.VMEM((3, B), jnp.float32),
            pltpu.VMEM((3, B, D), jnp.float32),
        ],
    )(logits, x, cu_seqlens)


@jax.jit
def _run(x, attention_query, cu_seqlens, attn_weight):
    q_t = attention_query.T
    logits = _sc_logits(x.reshape(N * D), q_t, attn_weight.reshape(GC * D))
    return _tc_pool(logits, x, cu_seqlens)


def kernel(x, attention_query, cu_seqlens, attn_weight):
    return (_run(x, attention_query, cu_seqlens, attn_weight), None, None)


# speedup vs baseline: 3.5504x; 3.5504x over previous
import jax
import jax.numpy as jnp
from jax import lax
from jax.experimental import pallas as pl
from jax.experimental.pallas import tpu as pltpu
from jax.experimental.pallas import tpu_sc as plsc

N = 32768
B = 16
D = 128
GC = 512
NEG = -1e30

NCORE = 2
NSUB = 16
NW = NCORE * NSUB
TPW = N // NW
CHT = 128
NCH = TPW // CHT
GRP = CHT // 16


def _sc_body(x_hbm, qt_hbm, w_hbm, out_hbm,
             w_v, xb0, xb1, qb0, qb1, lb, stg, sx0, sx1, sq0, sq1):
    wid = lax.axis_index("s") * NCORE + lax.axis_index("c")
    tok0 = wid * TPW

    pltpu.sync_copy(w_hbm, w_v)

    xbufs = (xb0, xb1)
    qbufs = (qb0, qb1)
    sxs = (sx0, sx1)
    sqs = (sq0, sq1)

    def issue(c, b):
        pltpu.async_copy(x_hbm.at[pl.ds((tok0 + c * CHT) * D, CHT * D)], xbufs[b], sxs[b])
        pltpu.async_copy(qt_hbm.at[:, pl.ds(tok0 + c * CHT, CHT)], qbufs[b], sqs[b])

    issue(0, 0)
    issue(1, 1)

    lane = jnp.arange(16, dtype=jnp.int32)
    dconst = [db * 16 + lane for db in range(D // 16)]
    tconst = [k + lane * 17 for k in range(16)]

    def outer(g, carry):
        for b in range(2):
            c = g * 2 + b
            pltpu.make_async_copy(x_hbm.at[pl.ds(0, CHT * D)], xbufs[b], sxs[b]).wait()
            pltpu.make_async_copy(qt_hbm.at[:, pl.ds(0, CHT)], qbufs[b], sqs[b]).wait()

            def grp_body(t, carry2, b=b, c=c):
                qvs = [qbufs[b][l, pl.ds(t * 16, 16)] for l in range(3)]
                for j in range(16):
                    bj = jnp.full((16,), j, jnp.int32)
                    qoff = [qvs[l].at[bj].get(mode="promise_in_bounds") * D
                            for l in range(3)]
                    base = t * (16 * D) + j * D
                    acc = [jnp.zeros((16,), jnp.float32) for _ in range(3)]
                    for db in range(D // 16):
                        xv = xbufs[b][pl.ds(base + db * 16, 16)]
                        for l in range(3):
                            wv = plsc.load_gather(w_v, [qoff[l] + dconst[db]])
                            acc[l] = acc[l] + xv * wv
                    for l in range(3):
                        stg[pl.ds(l * 272 + j * 17, 16)] = acc[l]
                for l in range(3):
                    red = jnp.zeros((16,), jnp.float32)
                    for k in range(16):
                        red = red + plsc.load_gather(stg, [l * 272 + tconst[k]])
                    lb[l, pl.ds(c * CHT + t * 16, 16)] = red
                return carry2

            lax.fori_loop(0, GRP, grp_body, 0)

            @pl.when(c + 2 < NCH)
            def _prefetch(b=b, c=c):
                issue(c + 2, b)
        return carry

    lax.fori_loop(0, NCH // 2, outer, 0)
    pltpu.sync_copy(lb, out_hbm.at[:, pl.ds(tok0, TPW)])


def _sc_logits(x, q_t, attn_weight):
    return pl.kernel(
        _sc_body,
        out_type=jax.ShapeDtypeStruct((3, N), jnp.float32),
        mesh=plsc.VectorSubcoreMesh(core_axis_name="c", subcore_axis_name="s"),
        compiler_params=pltpu.CompilerParams(needs_layout_passes=False),
        scratch_types=[
            pltpu.VMEM((GC * D,), jnp.float32),
            pltpu.VMEM((CHT * D,), jnp.float32),
            pltpu.VMEM((CHT * D,), jnp.float32),
            pltpu.VMEM((3, CHT), jnp.int32),
            pltpu.VMEM((3, CHT), jnp.int32),
            pltpu.VMEM((3, TPW), jnp.float32),
            pltpu.VMEM((3 * 272,), jnp.float32),
            pltpu.SemaphoreType.DMA,
            pltpu.SemaphoreType.DMA,
            pltpu.SemaphoreType.DMA,
            pltpu.SemaphoreType.DMA,
        ],
    )(x, q_t, attn_weight)


CH = 2048
NCHUNK = N // CH


def _tc_body(lg_ref, x_ref, cu_ref, out_ref, m_ref, d_ref, a_ref):
    i = pl.program_id(0)

    @pl.when(i == 0)
    def _init():
        m_ref[...] = jnp.full((3, B), NEG, jnp.float32)
        d_ref[...] = jnp.zeros((3, B), jnp.float32)
        a_ref[...] = jnp.zeros((3, B, D), jnp.float32)

    x_c = x_ref[...]
    tok = i * CH + lax.broadcasted_iota(jnp.int32, (1, CH), 1)
    cu = jnp.stack([cu_ref[bb] for bb in range(B + 1)])
    lo = cu[:B][:, None]
    hi = cu[1:][:, None]
    mask = (tok >= lo) & (tok < hi)
    lg = lg_ref[...]

    for l in range(3):
        logit = lg[l:l + 1, :]
        lmask = jnp.where(mask, logit, NEG)
        cmax = jnp.max(lmask, axis=1)
        m_old = m_ref[l, :]
        m_new = jnp.maximum(m_old, cmax)
        scale = jnp.exp(m_old - m_new)
        e = jnp.where(mask, jnp.exp(logit - m_new[:, None]), 0.0)
        d_new = d_ref[l, :] * scale + jnp.sum(e, axis=1)
        contrib = jnp.dot(e, x_c, preferred_element_type=jnp.float32)
        a_new = a_ref[l] * scale[:, None] + contrib
        m_ref[l, :] = m_new
        d_ref[l, :] = d_new
        a_ref[l] = a_new

        @pl.when(i == NCHUNK - 1)
        def _fin():
            denom = d_new[:, None]
            out_ref[l] = jnp.where(denom > 0.0, a_new / denom, 0.0)


def _tc_pool(logits, x, cu_seqlens):
    return pl.pallas_call(
        _tc_body,
        grid=(NCHUNK,),
        in_specs=[
            pl.BlockSpec((3, CH), lambda i: (0, i)),
            pl.BlockSpec((CH, D), lambda i: (i, 0)),
            pl.BlockSpec(memory_space=pltpu.SMEM),
        ],
        out_specs=pl.BlockSpec((3, B, D), lambda i: (0, 0, 0)),
        out_shape=jax.ShapeDtypeStruct((3, B, D), jnp.float32),
        scratch_shapes=[
            pltpu.VMEM((3, B), jnp.float32),
            pltpu.VMEM((3, B), jnp.float32),
            pltpu.VMEM((3, B, D), jnp.float32),
        ],
    )(logits, x, cu_seqlens)


@jax.jit
def _run(x, attention_query, cu_seqlens, attn_weight):
    q_t = attention_query.T
    logits = _sc_logits(x.reshape(N * D), q_t, attn_weight.reshape(GC * D))
    return _tc_pool(logits, x, cu_seqlens)


def kernel(x, attention_query, cu_seqlens, attn_weight):
    return (_run(x, attention_query, cu_seqlens, attn_weight), None, None)
